# baseline (speedup 1.0000x reference)
import jax
import jax.numpy as jnp
from jax import lax
from jax.experimental import pallas as pl
from jax.experimental.pallas import tpu as pltpu
from jax.experimental.pallas import tpu_sc as plsc

F32 = jnp.float32
NC = 2
NS = 16
LN = 16
CH = 128
PAD = 16
WB = 80


def _zero_vmem(ref, rows, cols):
    def body(q, _):
        i = q // (cols // LN)
        j = q % (cols // LN)
        ref[i, pl.ds(j * LN, LN)] = jnp.zeros((LN,), F32)
        return 0
    lax.fori_loop(0, rows * (cols // LN), body, 0)


def _sc_degree(dst2d, n):
    n_chunks = dst2d.shape[0]
    kt = n_chunks // (NC * NS)
    w_chunks = n // WB

    def body(dst_hbm, out_hbm, ones_v, zbuf, dst_v, acc, sem):
        c = lax.axis_index("c")
        s = lax.axis_index("s")
        w = c * NS + s

        def fill_ones(q, _):
            ones_v[q, :] = jnp.ones((LN,), F32)
            return 0
        lax.fori_loop(0, CH, fill_ones, 0)
        _zero_vmem(zbuf, WB, LN)

        pltpu.sync_copy(dst_hbm.at[pl.ds(w * kt, kt)], dst_v)

        def zero_acc(k, _):
            rc = k * NS + s
            @pl.when(rc < w_chunks)
            def _():
                pltpu.sync_copy(zbuf, acc.at[pl.ds(rc * WB, WB)])
            return 0
        lax.fori_loop(0, pl.cdiv(w_chunks, NS), zero_acc, 0)
        plsc.subcore_barrier()

        def edge_step(k, _):
            pltpu.sync_copy(ones_v, acc.at[dst_v.at[k]], add=True)
            return 0
        lax.fori_loop(0, kt, edge_step, 0)
        plsc.subcore_barrier()

        def write_step(k, _):
            rc = k * NS + s
            @pl.when(rc < w_chunks)
            def _():
                pltpu.sync_copy(acc.at[pl.ds(rc * WB, WB)],
                                out_hbm.at[pl.ds(c * n + rc * WB, WB)])
            return 0
        lax.fori_loop(0, pl.cdiv(w_chunks, NS), write_step, 0)

    mesh = plsc.VectorSubcoreMesh(core_axis_name="c", subcore_axis_name="s")
    f = pl.kernel(
        body,
        out_type=jax.ShapeDtypeStruct((2 * n, LN), F32),
        mesh=mesh,
        scratch_types=[
            pltpu.VMEM((CH, LN), F32),
            pltpu.VMEM((WB, LN), F32),
            pltpu.VMEM((kt, CH), jnp.int32),
            pltpu.VMEM_SHARED((n + PAD, LN), F32),
            pltpu.SemaphoreType.DMA,
        ],
    )
    return f(dst2d)


def _sc_aggregate(y2d, src2d, dst2d, n, h):
    n_chunks = src2d.shape[0]
    kt = n_chunks // NS
    n_phases = 2
    kph = kt // n_phases
    pairs = kph // 2
    w_chunks = n // WB

    def body(y_hbm, src_hbm, dst_hbm, out_hbm,
             src_v, dst_v, rows0, rows1, acc, sem0, sem1):
        c = lax.axis_index("c")
        s = lax.axis_index("s")
        cn = c * n

        def stage_indices(q):
            pltpu.sync_copy(src_hbm.at[pl.ds(s * kt + q * kph, kph)], src_v)
            pltpu.sync_copy(dst_hbm.at[pl.ds(s * kt + q * kph, kph)], dst_v)

            def off(t, _):
                i = t // (CH // LN)
                j = t % (CH // LN)
                src_v[i, pl.ds(j * LN, LN)] = src_v[i, pl.ds(j * LN, LN)] + cn
                return 0
            lax.fori_loop(0, kph * (CH // LN), off, 0)

        stage_indices(0)
        pltpu.async_copy(y_hbm.at[src_v.at[0]], rows0, sem0)

        def init_acc(k, _):
            rc = k * NS + s
            @pl.when(rc < w_chunks)
            def _():
                pltpu.sync_copy(y_hbm.at[pl.ds(cn + rc * WB, WB)],
                                acc.at[pl.ds(rc * WB, WB)])
            return 0
        lax.fori_loop(0, pl.cdiv(w_chunks, NS), init_acc, 0)

        @pl.when(s == 0)
        def _():
            def zrow(t, _):
                i = t // (h // LN)
                j = t % (h // LN)
                rows1[i, pl.ds(j * LN, LN)] = jnp.zeros((LN,), F32)
                return 0
            lax.fori_loop(0, PAD * (h // LN), zrow, 0)
            pltpu.sync_copy(rows1.at[pl.ds(0, PAD)], acc.at[pl.ds(n, PAD)])
        plsc.subcore_barrier()

        def pair_step(p, _):
            a = 2 * p
            b = 2 * p + 1
            pltpu.async_copy(y_hbm.at[src_v.at[b]], rows1, sem1)
            pltpu.make_async_copy(y_hbm.at[src_v.at[a]], rows0, sem0).wait()
            pltpu.sync_copy(rows0, acc.at[dst_v.at[a]], add=True)

            @pl.when(p < pairs - 1)
            def _():
                pltpu.async_copy(y_hbm.at[src_v.at[a + 2]], rows0, sem0)
            pltpu.make_async_copy(y_hbm.at[src_v.at[b]], rows1, sem1).wait()
            pltpu.sync_copy(rows1, acc.at[dst_v.at[b]], add=True)
            return 0

        for q in range(n_phases):
            if q > 0:
                stage_indices(q)
                pltpu.async_copy(y_hbm.at[src_v.at[0]], rows0, sem0)
            lax.fori_loop(0, pairs, pair_step, 0)
        plsc.subcore_barrier()

        def write_step(k, _):
            rc = k * NS + s
            @pl.when(rc < w_chunks)
            def _():
                pltpu.sync_copy(acc.at[pl.ds(rc * WB, WB)],
                                out_hbm.at[pl.ds(cn + rc * WB, WB)])
            return 0
        lax.fori_loop(0, pl.cdiv(w_chunks, NS), write_step, 0)

    mesh = plsc.VectorSubcoreMesh(core_axis_name="c", subcore_axis_name="s")
    f = pl.kernel(
        body,
        out_type=jax.ShapeDtypeStruct((2 * n, h), F32),
        mesh=mesh,
        scratch_types=[
            pltpu.VMEM((kt // n_phases, CH), jnp.int32),
            pltpu.VMEM((kt // n_phases, CH), jnp.int32),
            pltpu.VMEM((CH, h), F32),
            pltpu.VMEM((CH, h), F32),
            pltpu.VMEM_SHARED((n + PAD, h), F32),
            pltpu.SemaphoreType.DMA,
            pltpu.SemaphoreType.DMA,
        ],
    )
    return f(y2d, src2d, dst2d)


def _tc_pre(degs, x, w, n, b_rows):
    f_in = x.shape[1]
    f_out = w.shape[1]
    hh = f_out // 2
    grid = (n // b_rows,)

    def body(degs_ref, x_ref, w_ref, dis_ref, y_ref):
        deg = degs_ref[0, :, 0:1] + degs_ref[1, :, 0:1] + 1.0
        dis = lax.rsqrt(deg)
        xw = jnp.dot(x_ref[...], w_ref[...], preferred_element_type=F32,
                     precision=lax.Precision.HIGHEST)
        dis_ref[...] = dis
        y_ref[0] = dis * xw[:, :hh]
        y_ref[1] = dis * xw[:, hh:]

    return pl.pallas_call(
        body,
        grid=grid,
        in_specs=[
            pl.BlockSpec((2, b_rows, LN), lambda i: (0, i, 0)),
            pl.BlockSpec((b_rows, f_in), lambda i: (i, 0)),
            pl.BlockSpec((f_in, f_out), lambda i: (0, 0)),
        ],
        out_specs=[
            pl.BlockSpec((b_rows, 1), lambda i: (i, 0)),
            pl.BlockSpec((2, b_rows, hh), lambda i: (0, i, 0)),
        ],
        out_shape=[
            jax.ShapeDtypeStruct((n, 1), F32),
            jax.ShapeDtypeStruct((2, n, hh), F32),
        ],
    )(degs, x, w)


def _tc_stats(aggy, dis, b, n, b_rows):
    hh = aggy.shape[2]
    f = 2 * hh
    grid = (n // b_rows,)

    def body(agg_ref, dis_ref, b_ref, t_ref, st_ref):
        i = pl.program_id(0)
        dis = dis_ref[...]
        aggc = jnp.concatenate([agg_ref[0], agg_ref[1]], axis=1)
        t = dis * aggc + b_ref[...]
        t_ref[...] = t

        @pl.when(i == 0)
        def _():
            st_ref[...] = jnp.zeros_like(st_ref)
        st_ref[0:1, :] += jnp.sum(t, axis=0, keepdims=True)
        st_ref[1:2, :] += jnp.sum(t * t, axis=0, keepdims=True)

    return pl.pallas_call(
        body,
        grid=grid,
        in_specs=[
            pl.BlockSpec((2, b_rows, hh), lambda i: (0, i, 0)),
            pl.BlockSpec((b_rows, 1), lambda i: (i, 0)),
            pl.BlockSpec((1, f), lambda i: (0, 0)),
        ],
        out_specs=[
            pl.BlockSpec((b_rows, f), lambda i: (i, 0)),
            pl.BlockSpec((2, f), lambda i: (0, 0)),
        ],
        out_shape=[
            jax.ShapeDtypeStruct((n, f), F32),
            jax.ShapeDtypeStruct((2, f), F32),
        ],
        compiler_params=pltpu.CompilerParams(
            dimension_semantics=("arbitrary",)),
    )(aggy, dis, b)


def _bn_prelu_l2(t, st_ref, g_ref, be_ref, a_ref, nf):
    mu = st_ref[0:1, :] / nf
    var = st_ref[1:2, :] / nf - mu * mu
    h = (t - mu) / jnp.sqrt(var + 1e-5) * g_ref[...] + be_ref[...]
    a = a_ref[0, 0]
    h = jnp.where(h >= 0, h, a * h)
    nrm = jnp.sqrt(jnp.sum(h * h, axis=1, keepdims=True))
    return h / jnp.maximum(nrm, 1e-12)


def _tc_post(t, st, g, be, a, dis, w, n, b_rows):
    f = t.shape[1]
    f_out = w.shape[1]
    hh = f_out // 2
    grid = (n // b_rows,)
    nf = float(n)

    def body(t_ref, st_ref, g_ref, be_ref, a_ref, dis_ref, w_ref, y_ref):
        h = _bn_prelu_l2(t_ref[...], st_ref, g_ref, be_ref, a_ref, nf)
        xw = jnp.dot(h, w_ref[...], preferred_element_type=F32,
                     precision=lax.Precision.HIGHEST)
        dis = dis_ref[...]
        y_ref[0] = dis * xw[:, :hh]
        y_ref[1] = dis * xw[:, hh:]

    return pl.pallas_call(
        body,
        grid=grid,
        in_specs=[
            pl.BlockSpec((b_rows, f), lambda i: (i, 0)),
            pl.BlockSpec((2, f), lambda i: (0, 0)),
            pl.BlockSpec((1, f), lambda i: (0, 0)),
            pl.BlockSpec((1, f), lambda i: (0, 0)),
            pl.BlockSpec((1, 1), lambda i: (0, 0)),
            pl.BlockSpec((b_rows, 1), lambda i: (i, 0)),
            pl.BlockSpec((f, f_out), lambda i: (0, 0)),
        ],
        out_specs=pl.BlockSpec((2, b_rows, hh), lambda i: (0, i, 0)),
        out_shape=jax.ShapeDtypeStruct((2, n, hh), F32),
    )(t, st, g, be, a, dis, w)


def _tc_final(t, st, g, be, a, batch_col, wf1, bf1, wo, bo, n, n_graphs, b_rows):
    f = t.shape[1]
    fc1 = wf1.shape[1]
    grid = (n // b_rows,)
    last = n // b_rows - 1
    nf = float(n)

    def body(t_ref, st_ref, g_ref, be_ref, a_ref, batch_ref,
             wf1_ref, bf1_ref, wo_ref, bo_ref, out_ref, pool_acc, cnt_acc):
        i = pl.program_id(0)
        h = _bn_prelu_l2(t_ref[...], st_ref, g_ref, be_ref, a_ref, nf)

        gids = lax.broadcasted_iota(jnp.int32, (b_rows, n_graphs), 1)
        onehot = (gids == batch_ref[...]).astype(F32)

        @pl.when(i == 0)
        def _():
            pool_acc[...] = jnp.zeros_like(pool_acc)
            cnt_acc[...] = jnp.zeros_like(cnt_acc)
        dn = (((0,), (0,)), ((), ()))
        pool_acc[...] += lax.dot_general(onehot, h, dimension_numbers=dn,
                                         preferred_element_type=F32,
                                         precision=lax.Precision.HIGHEST)
        cnt_acc[...] += lax.dot_general(onehot, jnp.ones((b_rows, 1), F32),
                                        dimension_numbers=dn,
                                        preferred_element_type=F32,
                                        precision=lax.Precision.HIGHEST)

        @pl.when(i == last)
        def _():
            pooled = pool_acc[...] / jnp.maximum(cnt_acc[...], 1.0)
            hf = jnp.dot(pooled, wf1_ref[...], preferred_element_type=F32,
                         precision=lax.Precision.HIGHEST) + bf1_ref[...]
            hf = jnp.maximum(hf, 0.0)
            out_ref[...] = jnp.dot(hf, wo_ref[...], preferred_element_type=F32,
                                   precision=lax.Precision.HIGHEST) + bo_ref[...]

    return pl.pallas_call(
        body,
        grid=grid,
        in_specs=[
            pl.BlockSpec((b_rows, f), lambda i: (i, 0)),
            pl.BlockSpec((2, f), lambda i: (0, 0)),
            pl.BlockSpec((1, f), lambda i: (0, 0)),
            pl.BlockSpec((1, f), lambda i: (0, 0)),
            pl.BlockSpec((1, 1), lambda i: (0, 0)),
            pl.BlockSpec((b_rows, 1), lambda i: (i, 0)),
            pl.BlockSpec((f, fc1), lambda i: (0, 0)),
            pl.BlockSpec((1, fc1), lambda i: (0, 0)),
            pl.BlockSpec((fc1, 1), lambda i: (0, 0)),
            pl.BlockSpec((1, 1), lambda i: (0, 0)),
        ],
        out_specs=pl.BlockSpec((n_graphs, 1), lambda i: (0, 0)),
        out_shape=jax.ShapeDtypeStruct((n_graphs, 1), F32),
        scratch_shapes=[
            pltpu.VMEM((n_graphs, f), F32),
            pltpu.VMEM((n_graphs, 1), F32),
        ],
        compiler_params=pltpu.CompilerParams(
            dimension_semantics=("arbitrary",)),
    )(t, st, g, be, a, batch_col, wf1, bf1, wo, bo)


def kernel(x, edge_index, batch, W1, b1, g1, be1, a1, W2, b2, g2, be2, a2,
           Wf1, bf1, Wo, bo):
    n, f_in = x.shape
    e = edge_index.shape[1]
    h1 = W1.shape[1]
    h2 = W2.shape[1]
    n_graphs = 64
    b_rows = 1000

    chunk_total = NC * NS * CH
    ep = ((e + chunk_total - 1) // chunk_total) * chunk_total
    pad = ep - e
    src2d = jnp.concatenate(
        [edge_index[0], jnp.zeros((pad,), jnp.int32)]).reshape(ep // CH, CH)
    dst2d = jnp.concatenate(
        [edge_index[1], n + (jnp.arange(pad, dtype=jnp.int32) % PAD)]
    ).reshape(ep // CH, CH)

    degs = _sc_degree(dst2d, n).reshape(2, n, LN)
    dis, y1 = _tc_pre(degs, x, W1, n, b_rows)
    aggy1 = _sc_aggregate(y1.reshape(2 * n, h1 // 2), src2d, dst2d, n, h1 // 2)
    t1, st1 = _tc_stats(aggy1.reshape(2, n, h1 // 2), dis,
                        b1.reshape(1, h1), n, b_rows)
    y2 = _tc_post(t1, st1, g1.reshape(1, h1), be1.reshape(1, h1),
                  a1.reshape(1, 1), dis, W2, n, b_rows)
    aggy2 = _sc_aggregate(y2.reshape(2 * n, h2 // 2), src2d, dst2d, n, h2 // 2)
    t2, st2 = _tc_stats(aggy2.reshape(2, n, h2 // 2), dis,
                        b2.reshape(1, h2), n, b_rows)
    out = _tc_final(t2, st2, g2.reshape(1, h2), be2.reshape(1, h2),
                    a2.reshape(1, 1), batch.reshape(n, 1),
                    Wf1, bf1.reshape(1, -1), Wo, bo.reshape(1, 1),
                    n, n_graphs, b_rows)
    return out
